# baseline (speedup 1.0000x reference)
import functools

import jax
import jax.numpy as jnp
from jax.experimental import pallas as pl
from jax.experimental.pallas import tpu as pltpu

N = 4096
D = 2048
BN = 512


def _body(x_ref, wr_ref, br_ref, wl_ref, bl_ref, wrt_ref, brt_ref, o_ref):
    x = x_ref[...]

    wr = wr_ref[...]
    logits = jnp.sum(x * wr, axis=1, keepdims=True) + br_ref[0, 0]
    p = jax.nn.sigmoid(logits)

    x16 = x.astype(jnp.bfloat16)
    left = jnp.dot(x16, wl_ref[...], preferred_element_type=jnp.float32)
    left = jax.nn.relu(left + bl_ref[...])
    right = jnp.dot(x16, wrt_ref[...], preferred_element_type=jnp.float32)
    right = jax.nn.relu(right + brt_ref[...])

    o_ref[...] = p * left + (1.0 - p) * right


@jax.jit
def kernel(x, W_router, b_router, W_left, b_left, W_right, b_right):
    wr = W_router.reshape(1, D)
    br = b_router.reshape(1, 1)
    bl = b_left.reshape(1, D)
    brt = b_right.reshape(1, D)
    wl16 = W_left.astype(jnp.bfloat16)
    wrt16 = W_right.astype(jnp.bfloat16)

    grid = (N // BN,)
    return pl.pallas_call(
        _body,
        grid=grid,
        in_specs=[
            pl.BlockSpec((BN, D), lambda i: (i, 0)),
            pl.BlockSpec((1, D), lambda i: (0, 0)),
            pl.BlockSpec(memory_space=pltpu.SMEM),
            pl.BlockSpec((D, D), lambda i: (0, 0)),
            pl.BlockSpec((1, D), lambda i: (0, 0)),
            pl.BlockSpec((D, D), lambda i: (0, 0)),
            pl.BlockSpec((1, D), lambda i: (0, 0)),
        ],
        out_specs=pl.BlockSpec((BN, D), lambda i: (i, 0)),
        out_shape=jax.ShapeDtypeStruct((N, D), jnp.float32),
    )(x, wr, br, wl16, bl, wrt16, brt)


# speedup vs baseline: 1.1338x; 1.1338x over previous
import functools

import jax
import jax.numpy as jnp
from jax.experimental import pallas as pl
from jax.experimental.pallas import tpu as pltpu

N = 4096
D = 2048
BN = 512


def _body(x_ref, wr_ref, br_ref, wl_ref, bl_ref, wrt_ref, brt_ref, o_ref):
    x = x_ref[...]

    wr = wr_ref[...]
    logits = jnp.sum(x * wr, axis=1, keepdims=True) + br_ref[0, 0]
    p = jax.nn.sigmoid(logits)

    x16 = x.astype(jnp.bfloat16)
    left = jnp.dot(x16, wl_ref[...], preferred_element_type=jnp.float32)
    left = jax.nn.relu(left + bl_ref[...])
    right = jnp.dot(x16, wrt_ref[...], preferred_element_type=jnp.float32)
    right = jax.nn.relu(right + brt_ref[...])

    o_ref[...] = p * left + (1.0 - p) * right


@jax.jit
def kernel(x, W_router, b_router, W_left, b_left, W_right, b_right):
    wr = W_router.reshape(1, D)
    br = b_router.reshape(1, 1)
    bl = b_left.reshape(1, D)
    brt = b_right.reshape(1, D)

    grid = (N // BN,)
    return pl.pallas_call(
        _body,
        grid=grid,
        in_specs=[
            pl.BlockSpec((BN, D), lambda i: (i, 0)),
            pl.BlockSpec((1, D), lambda i: (0, 0)),
            pl.BlockSpec(memory_space=pltpu.SMEM),
            pl.BlockSpec((D, D), lambda i: (0, 0)),
            pl.BlockSpec((1, D), lambda i: (0, 0)),
            pl.BlockSpec((D, D), lambda i: (0, 0)),
            pl.BlockSpec((1, D), lambda i: (0, 0)),
        ],
        out_specs=pl.BlockSpec((BN, D), lambda i: (i, 0)),
        out_shape=jax.ShapeDtypeStruct((N, D), jnp.float32),
    )(x, wr, br, W_left, bl, W_right, brt)
